# Initial kernel scaffold; baseline (speedup 1.0000x reference)
import jax
import jax.numpy as jnp
from jax.experimental import pallas as pl


def kernel(edge_index, nfeats, efeats, W_msg, b_msg, W_apply, b_apply):
    raise NotImplementedError("write your pallas kernel here")


# speedup vs baseline: 1.0295x; 1.0295x over previous
import functools

import jax
import jax.numpy as jnp
from jax import lax
from jax.experimental import pallas as pl
from jax.experimental.pallas import tpu as pltpu

N = 10000
E = 320000
DIN = 128
DE = 16
DOUT = 128


def _x1_body(x_ref, w_ref, o_ref):
    o_ref[...] = jnp.dot(x_ref[...], w_ref[...],
                         preferred_element_type=jnp.float32)


def _e2_body(e_ref, w_ref, b_ref, o_ref):
    o_ref[...] = jnp.dot(e_ref[...], w_ref[...],
                         preferred_element_type=jnp.float32) + b_ref[...]


def _apply_body(x_ref, h0_ref, h1_ref, wa1_ref, wa2_ref, b_ref, o_ref):
    acc = jnp.dot(x_ref[...], wa1_ref[...], preferred_element_type=jnp.float32)
    h = h0_ref[0] + h1_ref[0]
    acc = acc + jnp.dot(h, wa2_ref[...], preferred_element_type=jnp.float32)
    o_ref[...] = jnp.maximum(acc + b_ref[...], 0.0)


def _tc_x1(x, w1):
    nb = 10
    blk = N // nb
    return pl.pallas_call(
        _x1_body,
        grid=(nb,),
        in_specs=[
            pl.BlockSpec((blk, DIN), lambda i: (i, 0)),
            pl.BlockSpec((DIN, DOUT), lambda i: (0, 0)),
        ],
        out_specs=pl.BlockSpec((blk, DOUT), lambda i: (i, 0)),
        out_shape=jax.ShapeDtypeStruct((N, DOUT), jnp.float32),
    )(x, w1)


def _tc_e2(e_pad, w2, b, e_pad_rows):
    nb = e_pad_rows // 2048
    return pl.pallas_call(
        _e2_body,
        grid=(nb,),
        in_specs=[
            pl.BlockSpec((2048, DE), lambda i: (i, 0)),
            pl.BlockSpec((DE, DOUT), lambda i: (0, 0)),
            pl.BlockSpec((1, DOUT), lambda i: (0, 0)),
        ],
        out_specs=pl.BlockSpec((2048, DOUT), lambda i: (i, 0)),
        out_shape=jax.ShapeDtypeStruct((e_pad_rows, DOUT), jnp.float32),
    )(e_pad, w2, b)


def _tc_apply(x, parts, wa1, wa2, b, n_pad):
    nb = 10
    blk = N // nb
    return pl.pallas_call(
        _apply_body,
        grid=(nb,),
        in_specs=[
            pl.BlockSpec((blk, DIN), lambda i: (i, 0)),
            pl.BlockSpec((1, blk, DOUT), lambda i: (0, i, 0)),
            pl.BlockSpec((1, blk, DOUT), lambda i: (1, i, 0)),
            pl.BlockSpec((DIN, DOUT), lambda i: (0, 0)),
            pl.BlockSpec((DOUT, DOUT), lambda i: (0, 0)),
            pl.BlockSpec((1, DOUT), lambda i: (0, 0)),
        ],
        out_specs=pl.BlockSpec((blk, DOUT), lambda i: (i, 0)),
        out_shape=jax.ShapeDtypeStruct((N, DOUT), jnp.float32),
    )(x, parts, parts, wa1, wa2, b)


def kernel(edge_index, nfeats, efeats, W_msg, b_msg, W_apply, b_apply):
    x = nfeats.reshape(N, DIN)
    e = efeats.reshape(E, DE)
    src = edge_index[0]
    dst = edge_index[1]

    W1 = W_msg[:DIN]
    W2 = W_msg[DIN:]
    Wa1 = W_apply[:DIN]
    Wa2 = W_apply[DIN:]

    x1 = _tc_x1(x, W1)
    e2 = _tc_e2(e, W2, b_msg.reshape(1, DOUT), E)

    m = jnp.maximum(jnp.take(x1, src, axis=0) + e2, 0.0)
    h = jax.ops.segment_sum(m, dst, num_segments=N)
    parts = jnp.stack([h, jnp.zeros_like(h)])

    out = _tc_apply(x, parts, Wa1, Wa2, b_apply.reshape(1, DOUT), N)
    return out.reshape(N, 1, DOUT)


# speedup vs baseline: 2.5036x; 2.4319x over previous
import functools

import jax
import jax.numpy as jnp
from jax import lax
from jax.experimental import pallas as pl
from jax.experimental.pallas import tpu as pltpu
from jax.experimental.pallas import tpu_sc as plsc

N = 10000
E = 320000
DIN = 128
DE = 16
DOUT = 128

NC = 2
NS = 16
NW = NC * NS
CHUNK = 128
CPT = 79
EPT = CHUNK * CPT
E_PAD = EPT * NW
N_PAD = 10112
RPT = N_PAD // NS


def _x1_body(x_ref, w_ref, o_ref):
    o_ref[...] = jnp.dot(x_ref[...], w_ref[...],
                         preferred_element_type=jnp.float32)


def _e2_body(e_ref, w_ref, b_ref, o_ref):
    o_ref[...] = jnp.dot(e_ref[...], w_ref[...],
                         preferred_element_type=jnp.float32) + b_ref[...]


def _apply_body(x_ref, h0_ref, h1_ref, wa1_ref, wa2_ref, b_ref, o_ref):
    acc = jnp.dot(x_ref[...], wa1_ref[...], preferred_element_type=jnp.float32)
    h = h0_ref[0] + h1_ref[0]
    acc = acc + jnp.dot(h, wa2_ref[...], preferred_element_type=jnp.float32)
    o_ref[...] = jnp.maximum(acc + b_ref[...], 0.0)


def _tc_x1(x, w1):
    nb = 10
    blk = N // nb
    return pl.pallas_call(
        _x1_body,
        grid=(nb,),
        in_specs=[
            pl.BlockSpec((blk, DIN), lambda i: (i, 0)),
            pl.BlockSpec((DIN, DOUT), lambda i: (0, 0)),
        ],
        out_specs=pl.BlockSpec((blk, DOUT), lambda i: (i, 0)),
        out_shape=jax.ShapeDtypeStruct((N, DOUT), jnp.float32),
    )(x, w1)


def _tc_e2(e_pad, w2, b, e_pad_rows):
    nb = e_pad_rows // 2048
    return pl.pallas_call(
        _e2_body,
        grid=(nb,),
        in_specs=[
            pl.BlockSpec((2048, DE), lambda i: (i, 0)),
            pl.BlockSpec((DE, DOUT), lambda i: (0, 0)),
            pl.BlockSpec((1, DOUT), lambda i: (0, 0)),
        ],
        out_specs=pl.BlockSpec((2048, DOUT), lambda i: (i, 0)),
        out_shape=jax.ShapeDtypeStruct((e_pad_rows, DOUT), jnp.float32),
    )(e_pad, w2, b)


def _tc_apply(x, parts, wa1, wa2, b, n_pad):
    nb = 10
    blk = N // nb
    return pl.pallas_call(
        _apply_body,
        grid=(nb,),
        in_specs=[
            pl.BlockSpec((blk, DIN), lambda i: (i, 0)),
            pl.BlockSpec((1, blk, DOUT), lambda i: (0, i, 0)),
            pl.BlockSpec((1, blk, DOUT), lambda i: (1, i, 0)),
            pl.BlockSpec((DIN, DOUT), lambda i: (0, 0)),
            pl.BlockSpec((DOUT, DOUT), lambda i: (0, 0)),
            pl.BlockSpec((1, DOUT), lambda i: (0, 0)),
        ],
        out_specs=pl.BlockSpec((blk, DOUT), lambda i: (i, 0)),
        out_shape=jax.ShapeDtypeStruct((N, DOUT), jnp.float32),
    )(x, parts, parts, wa1, wa2, b)


def _sc_body(x1_hbm, e2_hbm, src_hbm, dst_hbm, zeros_hbm, parts_hbm,
             src_v, dst_v, rows_v, e2_v, acc_sh, sem):
    c = lax.axis_index("c")
    s = lax.axis_index("s")
    wid = s * NC + c

    pltpu.sync_copy(zeros_hbm.at[pl.ds(s * RPT, RPT)],
                    acc_sh.at[pl.ds(s * RPT, RPT)])
    plsc.subcore_barrier()

    def chunk_body(i, _):
        off = wid * EPT + i * CHUNK
        pltpu.sync_copy(src_hbm.at[pl.ds(off, CHUNK)], src_v)
        gat = pltpu.async_copy(x1_hbm.at[src_v], rows_v, sem)
        pltpu.sync_copy(dst_hbm.at[pl.ds(off, CHUNK)], dst_v)
        pltpu.sync_copy(e2_hbm.at[pl.ds(off, CHUNK)], e2_v)
        gat.wait()

        def row_body(r, _):
            for j in range(8):
                sl = pl.ds(j * 16, 16)
                rows_v[r, sl] = jnp.maximum(rows_v[r, sl] + e2_v[r, sl], 0.0)
            return 0

        lax.fori_loop(0, CHUNK, row_body, 0)
        pltpu.sync_copy(rows_v, acc_sh.at[dst_v], add=True)
        return 0

    lax.fori_loop(0, CPT, chunk_body, 0)
    plsc.subcore_barrier()

    pltpu.sync_copy(acc_sh.at[pl.ds(s * RPT, RPT)],
                    parts_hbm.at[c, pl.ds(s * RPT, RPT)])


def _sc_scatter(x1, e2p, srcp, dstp, zeros):
    mesh = plsc.VectorSubcoreMesh(core_axis_name="c", subcore_axis_name="s")
    f = pl.kernel(
        _sc_body,
        out_type=jax.ShapeDtypeStruct((NC, N_PAD, DOUT), jnp.float32),
        mesh=mesh,
        scratch_types=[
            pltpu.VMEM((CHUNK,), jnp.int32),
            pltpu.VMEM((CHUNK,), jnp.int32),
            pltpu.VMEM((CHUNK, DOUT), jnp.float32),
            pltpu.VMEM((CHUNK, DOUT), jnp.float32),
            pltpu.VMEM_SHARED((N_PAD, DOUT), jnp.float32),
            pltpu.SemaphoreType.DMA,
        ],
    )
    return f(x1, e2p, srcp, dstp, zeros)


def kernel(edge_index, nfeats, efeats, W_msg, b_msg, W_apply, b_apply):
    x = nfeats.reshape(N, DIN)
    e = efeats.reshape(E, DE)
    src = edge_index[0]
    dst = edge_index[1]

    W1 = W_msg[:DIN]
    W2 = W_msg[DIN:]
    Wa1 = W_apply[:DIN]
    Wa2 = W_apply[DIN:]

    x1 = _tc_x1(x, W1)

    e_pad = jnp.pad(e, ((0, E_PAD - E), (0, 0)))
    srcp = jnp.pad(src, (0, E_PAD - E))
    dstp = jnp.pad(dst, (0, E_PAD - E), constant_values=N)
    zeros = jnp.zeros((N_PAD, DOUT), jnp.float32)

    e2p = _tc_e2(e_pad, W2, b_msg.reshape(1, DOUT), E_PAD)
    parts = _sc_scatter(x1, e2p, srcp, dstp, zeros)

    out = _tc_apply(x, parts, Wa1, Wa2, b_apply.reshape(1, DOUT), N)
    return out.reshape(N, 1, DOUT)


# speedup vs baseline: 2.9516x; 1.1789x over previous
import functools

import jax
import jax.numpy as jnp
from jax import lax
from jax.experimental import pallas as pl
from jax.experimental.pallas import tpu as pltpu
from jax.experimental.pallas import tpu_sc as plsc

N = 10000
E = 320000
DIN = 128
DE = 16
DOUT = 128

NC = 2
NS = 16
NW = NC * NS
CHUNK = 64
CPT = 158
EPT = CHUNK * CPT
E_PAD = EPT * NW
NCHT = E_PAD // CHUNK
N_PAD = 10112
RPT = N_PAD // NS


def _x1_body(x_ref, w_ref, o_ref):
    o_ref[...] = jnp.dot(x_ref[...], w_ref[...],
                         preferred_element_type=jnp.float32)


def _e2_body(e_ref, w_ref, b_ref, o_ref):
    o_ref[...] = jnp.dot(e_ref[...], w_ref[...],
                         preferred_element_type=jnp.float32) + b_ref[...]


def _apply_body(x_ref, h0_ref, h1_ref, wa1_ref, wa2_ref, b_ref, o_ref):
    acc = jnp.dot(x_ref[...], wa1_ref[...], preferred_element_type=jnp.float32)
    h = h0_ref[0] + h1_ref[0]
    acc = acc + jnp.dot(h, wa2_ref[...], preferred_element_type=jnp.float32)
    o_ref[...] = jnp.maximum(acc + b_ref[...], 0.0)


def _tc_x1(x, w1):
    nb = 10
    blk = N // nb
    return pl.pallas_call(
        _x1_body,
        grid=(nb,),
        in_specs=[
            pl.BlockSpec((blk, DIN), lambda i: (i, 0)),
            pl.BlockSpec((DIN, DOUT), lambda i: (0, 0)),
        ],
        out_specs=pl.BlockSpec((blk, DOUT), lambda i: (i, 0)),
        out_shape=jax.ShapeDtypeStruct((N, DOUT), jnp.float32),
    )(x, w1)


def _tc_e2(e_pad, w2, b):
    nb = E_PAD // 2048
    return pl.pallas_call(
        _e2_body,
        grid=(nb,),
        in_specs=[
            pl.BlockSpec((2048, DE), lambda i: (i, 0)),
            pl.BlockSpec((DE, DOUT), lambda i: (0, 0)),
            pl.BlockSpec((1, DOUT), lambda i: (0, 0)),
        ],
        out_specs=pl.BlockSpec((2048, DOUT), lambda i: (i, 0)),
        out_shape=jax.ShapeDtypeStruct((E_PAD, DOUT), jnp.float32),
    )(e_pad, w2, b)


def _tc_apply(x, parts, wa1, wa2, b):
    nb = 10
    blk = N // nb
    return pl.pallas_call(
        _apply_body,
        grid=(nb,),
        in_specs=[
            pl.BlockSpec((blk, DIN), lambda i: (i, 0)),
            pl.BlockSpec((1, blk, DOUT), lambda i: (0, i, 0)),
            pl.BlockSpec((1, blk, DOUT), lambda i: (1, i, 0)),
            pl.BlockSpec((DIN, DOUT), lambda i: (0, 0)),
            pl.BlockSpec((DOUT, DOUT), lambda i: (0, 0)),
            pl.BlockSpec((1, DOUT), lambda i: (0, 0)),
        ],
        out_specs=pl.BlockSpec((blk, DOUT), lambda i: (i, 0)),
        out_shape=jax.ShapeDtypeStruct((N, DOUT), jnp.float32),
    )(x, parts, parts, wa1, wa2, b)


def _sc_body(x1_hbm, e2_hbm, idx_hbm, zeros_hbm, parts_hbm,
             idx0, idx1, rows0, rows1, e20, e21, sb0, sb1, dstx0, dstx1,
             acc_sh,
             lsem0, lsem1, gsem0, gsem1, esem0, esem1, ssem0, ssem1):
    c = lax.axis_index("c")
    s = lax.axis_index("s")
    wid = s * NC + c
    idxb = (idx0, idx1)
    rows = (rows0, rows1)
    e2b = (e20, e21)
    sb = (sb0, sb1)
    dstx = (dstx0, dstx1)
    lsem = (lsem0, lsem1)
    gsem = (gsem0, gsem1)
    esem = (esem0, esem1)
    ssem = (ssem0, ssem1)

    cbase = wid * CPT

    pltpu.sync_copy(zeros_hbm.at[pl.ds(s * RPT, RPT)],
                    acc_sh.at[pl.ds(s * RPT, RPT)])
    plsc.subcore_barrier()

    def issue_idx(p, i):
        pltpu.make_async_copy(idx_hbm.at[cbase + i], idxb[p], lsem[p]).start()

    def wait_idx(p):
        pltpu.make_async_copy(idx_hbm.at[0], idxb[p], lsem[p]).wait()

    def issue_gather(p):
        pltpu.make_async_copy(x1_hbm.at[idxb[p].at[0]], rows[p],
                              gsem[p]).start()

    def wait_gather(p):
        pltpu.make_async_copy(x1_hbm.at[idxb[p].at[0]], rows[p],
                              gsem[p]).wait()

    def issue_e2(p, i):
        off = (cbase + i) * CHUNK
        pltpu.make_async_copy(e2_hbm.at[pl.ds(off, CHUNK)], e2b[p],
                              esem[p]).start()

    def wait_e2(p):
        pltpu.make_async_copy(e2_hbm.at[pl.ds(0, CHUNK)], e2b[p],
                              esem[p]).wait()

    def issue_scatter(p):
        pltpu.async_copy(sb[p], acc_sh.at[dstx[p]], ssem[p], add=True)

    def wait_scatter(p):
        pltpu.make_async_copy(sb[p], acc_sh.at[dstx[p]], ssem[p]).wait()

    def step(p, i):
        p1 = 1 - p
        wait_gather(p)

        @pl.when(i + 1 < CPT)
        def _():
            wait_idx(p1)
            issue_gather(p1)
            issue_e2(p1, i + 1)

        wait_e2(p)

        @pl.when(i >= 2)
        def _():
            wait_scatter(p)

        def cbody(r):
            for j in range(8):
                sl = pl.ds(j * 16, 16)
                sb[p][r, sl] = jnp.maximum(rows[p][r, sl] + e2b[p][r, sl], 0.0)
        plsc.parallel_loop(0, CHUNK, 1, unroll=4)(cbody)

        for j in range(4):
            sl = pl.ds(j * 16, 16)
            dstx[p][sl] = idxb[p][1, sl]
        issue_scatter(p)

        @pl.when(i + 2 < CPT)
        def _():
            issue_idx(p, i + 2)

    issue_idx(0, 0)
    issue_idx(1, 1)
    wait_idx(0)
    issue_gather(0)
    issue_e2(0, 0)

    def pair_body(k, _):
        step(0, 2 * k)
        step(1, 2 * k + 1)
        return 0

    lax.fori_loop(0, CPT // 2, pair_body, 0)
    wait_scatter(0)
    wait_scatter(1)
    plsc.subcore_barrier()

    pltpu.sync_copy(acc_sh.at[pl.ds(s * RPT, RPT)],
                    parts_hbm.at[c, pl.ds(s * RPT, RPT)])


def _sc_scatter(x1, e2p, idx_pairs, zeros):
    mesh = plsc.VectorSubcoreMesh(core_axis_name="c", subcore_axis_name="s")
    f = pl.kernel(
        _sc_body,
        out_type=jax.ShapeDtypeStruct((NC, N_PAD, DOUT), jnp.float32),
        mesh=mesh,
        scratch_types=[
            pltpu.VMEM((2, CHUNK), jnp.int32),
            pltpu.VMEM((2, CHUNK), jnp.int32),
            pltpu.VMEM((CHUNK, DOUT), jnp.float32),
            pltpu.VMEM((CHUNK, DOUT), jnp.float32),
            pltpu.VMEM((CHUNK, DOUT), jnp.float32),
            pltpu.VMEM((CHUNK, DOUT), jnp.float32),
            pltpu.VMEM((CHUNK, DOUT), jnp.float32),
            pltpu.VMEM((CHUNK, DOUT), jnp.float32),
            pltpu.VMEM((CHUNK,), jnp.int32),
            pltpu.VMEM((CHUNK,), jnp.int32),
            pltpu.VMEM_SHARED((N_PAD, DOUT), jnp.float32),
            pltpu.SemaphoreType.DMA,
            pltpu.SemaphoreType.DMA,
            pltpu.SemaphoreType.DMA,
            pltpu.SemaphoreType.DMA,
            pltpu.SemaphoreType.DMA,
            pltpu.SemaphoreType.DMA,
            pltpu.SemaphoreType.DMA,
            pltpu.SemaphoreType.DMA,
        ],
    )
    return f(x1, e2p, idx_pairs, zeros)


def kernel(edge_index, nfeats, efeats, W_msg, b_msg, W_apply, b_apply):
    x = nfeats.reshape(N, DIN)
    e = efeats.reshape(E, DE)
    src = edge_index[0]
    dst = edge_index[1]

    W1 = W_msg[:DIN]
    W2 = W_msg[DIN:]
    Wa1 = W_apply[:DIN]
    Wa2 = W_apply[DIN:]

    x1 = _tc_x1(x, W1)

    e_pad = jnp.pad(e, ((0, E_PAD - E), (0, 0)))
    srcp = jnp.pad(src, (0, E_PAD - E))
    dstp = jnp.pad(dst, (0, E_PAD - E), constant_values=N)
    idx_pairs = jnp.stack([srcp.reshape(NCHT, CHUNK),
                           dstp.reshape(NCHT, CHUNK)], axis=1)
    zeros = jnp.zeros((N_PAD, DOUT), jnp.float32)

    e2p = _tc_e2(e_pad, W2, b_msg.reshape(1, DOUT))
    parts = _sc_scatter(x1, e2p, idx_pairs, zeros)

    out = _tc_apply(x, parts, Wa1, Wa2, b_apply.reshape(1, DOUT))
    return out.reshape(N, 1, DOUT)


# speedup vs baseline: 3.1728x; 1.0749x over previous
import functools

import jax
import jax.numpy as jnp
from jax import lax
from jax.experimental import pallas as pl
from jax.experimental.pallas import tpu as pltpu
from jax.experimental.pallas import tpu_sc as plsc

N = 10000
E = 320000
DIN = 128
DE = 16
DOUT = 128

NC = 2
NS = 16
NW = NC * NS
CHUNK = 64
CPT = 158
EPT = CHUNK * CPT
E_PAD = EPT * NW
NCHT = E_PAD // CHUNK
N_PAD = 10112
RPT = N_PAD // NS


def _x1_body(x_ref, w_ref, o_ref):
    o_ref[...] = jnp.dot(x_ref[...], w_ref[...],
                         preferred_element_type=jnp.float32)


def _e2_body(e_ref, w_ref, b_ref, o_ref):
    o_ref[...] = jnp.dot(e_ref[...], w_ref[...],
                         preferred_element_type=jnp.float32) + b_ref[...]


def _apply_body(x_ref, h0_ref, h1_ref, wa1_ref, wa2_ref, b_ref, o_ref):
    acc = jnp.dot(x_ref[...], wa1_ref[...], preferred_element_type=jnp.float32)
    h = h0_ref[0] + h1_ref[0]
    acc = acc + jnp.dot(h, wa2_ref[...], preferred_element_type=jnp.float32)
    o_ref[...] = jnp.maximum(acc + b_ref[...], 0.0)


def _tc_x1(x, w1):
    nb = 10
    blk = N // nb
    return pl.pallas_call(
        _x1_body,
        grid=(nb,),
        in_specs=[
            pl.BlockSpec((blk, DIN), lambda i: (i, 0)),
            pl.BlockSpec((DIN, DOUT), lambda i: (0, 0)),
        ],
        out_specs=pl.BlockSpec((blk, DOUT), lambda i: (i, 0)),
        out_shape=jax.ShapeDtypeStruct((N, DOUT), jnp.float32),
    )(x, w1)


def _tc_e2(e, w2, b):
    blk = 2000
    nb = E // blk
    return pl.pallas_call(
        _e2_body,
        grid=(nb,),
        in_specs=[
            pl.BlockSpec((blk, DE), lambda i: (i, 0)),
            pl.BlockSpec((DE, DOUT), lambda i: (0, 0)),
            pl.BlockSpec((1, DOUT), lambda i: (0, 0)),
        ],
        out_specs=pl.BlockSpec((blk, DOUT), lambda i: (i, 0)),
        out_shape=jax.ShapeDtypeStruct((E_PAD, DOUT), jnp.float32),
    )(e, w2, b)


def _tc_apply(x, parts, wa1, wa2, b):
    nb = 10
    blk = N // nb
    return pl.pallas_call(
        _apply_body,
        grid=(nb,),
        in_specs=[
            pl.BlockSpec((blk, DIN), lambda i: (i, 0)),
            pl.BlockSpec((1, blk, DOUT), lambda i: (0, i, 0)),
            pl.BlockSpec((1, blk, DOUT), lambda i: (1, i, 0)),
            pl.BlockSpec((DIN, DOUT), lambda i: (0, 0)),
            pl.BlockSpec((DOUT, DOUT), lambda i: (0, 0)),
            pl.BlockSpec((1, DOUT), lambda i: (0, 0)),
        ],
        out_specs=pl.BlockSpec((blk, DOUT), lambda i: (i, 0)),
        out_shape=jax.ShapeDtypeStruct((N, DOUT), jnp.float32),
    )(x, parts, parts, wa1, wa2, b)


def _sc_body(x1_hbm, e2_hbm, idx_hbm, zeros_hbm, parts_hbm,
             idx0, idx1, rows0, rows1, e20, e21, sb0, sb1, dstx0, dstx1,
             acc_sh,
             lsem0, lsem1, gsem0, gsem1, esem0, esem1, ssem0, ssem1):
    c = lax.axis_index("c")
    s = lax.axis_index("s")
    wid = s * NC + c
    idxb = (idx0, idx1)
    rows = (rows0, rows1)
    e2b = (e20, e21)
    sb = (sb0, sb1)
    dstx = (dstx0, dstx1)
    lsem = (lsem0, lsem1)
    gsem = (gsem0, gsem1)
    esem = (esem0, esem1)
    ssem = (ssem0, ssem1)

    cbase = wid * CPT

    pltpu.sync_copy(zeros_hbm.at[pl.ds(s * RPT, RPT)],
                    acc_sh.at[pl.ds(s * RPT, RPT)])
    plsc.subcore_barrier()

    def issue_idx(p, i):
        pltpu.make_async_copy(idx_hbm.at[cbase + i], idxb[p], lsem[p]).start()

    def wait_idx(p):
        pltpu.make_async_copy(idx_hbm.at[0], idxb[p], lsem[p]).wait()

    def issue_gather(p):
        pltpu.make_async_copy(x1_hbm.at[idxb[p].at[0]], rows[p],
                              gsem[p]).start()

    def wait_gather(p):
        pltpu.make_async_copy(x1_hbm.at[idxb[p].at[0]], rows[p],
                              gsem[p]).wait()

    def issue_e2(p, i):
        off = (cbase + i) * CHUNK
        pltpu.make_async_copy(e2_hbm.at[pl.ds(off, CHUNK)], e2b[p],
                              esem[p]).start()

    def wait_e2(p):
        pltpu.make_async_copy(e2_hbm.at[pl.ds(0, CHUNK)], e2b[p],
                              esem[p]).wait()

    def issue_scatter(p):
        pltpu.async_copy(sb[p], acc_sh.at[dstx[p]], ssem[p], add=True)

    def wait_scatter(p):
        pltpu.make_async_copy(sb[p], acc_sh.at[dstx[p]], ssem[p]).wait()

    def step(p, i):
        p1 = 1 - p
        wait_gather(p)

        @pl.when(i + 1 < CPT)
        def _():
            wait_idx(p1)
            issue_gather(p1)
            issue_e2(p1, i + 1)

        wait_e2(p)

        @pl.when(i >= 2)
        def _():
            wait_scatter(p)

        def cbody(r):
            for j in range(8):
                sl = pl.ds(j * 16, 16)
                sb[p][r, sl] = jnp.maximum(rows[p][r, sl] + e2b[p][r, sl], 0.0)
        plsc.parallel_loop(0, CHUNK, 1, unroll=4)(cbody)

        for j in range(4):
            sl = pl.ds(j * 16, 16)
            dstx[p][sl] = idxb[p][1, sl]
        issue_scatter(p)

        @pl.when(i + 2 < CPT)
        def _():
            issue_idx(p, i + 2)

    issue_idx(0, 0)
    issue_idx(1, 1)
    wait_idx(0)
    issue_gather(0)
    issue_e2(0, 0)

    def pair_body(k, _):
        step(0, 2 * k)
        step(1, 2 * k + 1)
        return 0

    lax.fori_loop(0, CPT // 2, pair_body, 0)
    wait_scatter(0)
    wait_scatter(1)
    plsc.subcore_barrier()

    pltpu.sync_copy(acc_sh.at[pl.ds(s * RPT, RPT)],
                    parts_hbm.at[c, pl.ds(s * RPT, RPT)])


def _sc_scatter(x1, e2p, idx_pairs, zeros):
    mesh = plsc.VectorSubcoreMesh(core_axis_name="c", subcore_axis_name="s")
    f = pl.kernel(
        _sc_body,
        out_type=jax.ShapeDtypeStruct((NC, N_PAD, DOUT), jnp.float32),
        mesh=mesh,
        scratch_types=[
            pltpu.VMEM((2, CHUNK), jnp.int32),
            pltpu.VMEM((2, CHUNK), jnp.int32),
            pltpu.VMEM((CHUNK, DOUT), jnp.float32),
            pltpu.VMEM((CHUNK, DOUT), jnp.float32),
            pltpu.VMEM((CHUNK, DOUT), jnp.float32),
            pltpu.VMEM((CHUNK, DOUT), jnp.float32),
            pltpu.VMEM((CHUNK, DOUT), jnp.float32),
            pltpu.VMEM((CHUNK, DOUT), jnp.float32),
            pltpu.VMEM((CHUNK,), jnp.int32),
            pltpu.VMEM((CHUNK,), jnp.int32),
            pltpu.VMEM_SHARED((N_PAD, DOUT), jnp.float32),
            pltpu.SemaphoreType.DMA,
            pltpu.SemaphoreType.DMA,
            pltpu.SemaphoreType.DMA,
            pltpu.SemaphoreType.DMA,
            pltpu.SemaphoreType.DMA,
            pltpu.SemaphoreType.DMA,
            pltpu.SemaphoreType.DMA,
            pltpu.SemaphoreType.DMA,
        ],
    )
    return f(x1, e2p, idx_pairs, zeros)


def kernel(edge_index, nfeats, efeats, W_msg, b_msg, W_apply, b_apply):
    x = nfeats.reshape(N, DIN)
    e = efeats.reshape(E, DE)
    src = edge_index[0]
    dst = edge_index[1]

    W1 = W_msg[:DIN]
    W2 = W_msg[DIN:]
    Wa1 = W_apply[:DIN]
    Wa2 = W_apply[DIN:]

    x1 = _tc_x1(x, W1)

    srcp = jnp.pad(src, (0, E_PAD - E))
    dstp = jnp.pad(dst, (0, E_PAD - E), constant_values=N)
    idx_pairs = jnp.stack([srcp.reshape(NCHT, CHUNK),
                           dstp.reshape(NCHT, CHUNK)], axis=1)
    zeros = jnp.zeros((N_PAD, DOUT), jnp.float32)

    e2p = _tc_e2(e.astype(jnp.bfloat16), W2.astype(jnp.bfloat16),
                 b_msg.reshape(1, DOUT))
    parts = _sc_scatter(x1, e2p, idx_pairs, zeros)

    out = _tc_apply(x, parts, Wa1, Wa2, b_apply.reshape(1, DOUT))
    return out.reshape(N, 1, DOUT)


# speedup vs baseline: 3.9025x; 1.2300x over previous
import functools

import jax
import jax.numpy as jnp
from jax import lax
from jax.experimental import pallas as pl
from jax.experimental.pallas import tpu as pltpu
from jax.experimental.pallas import tpu_sc as plsc

N = 10000
E = 320000
DIN = 128
DE = 16
DOUT = 128

NC = 2
NS = 16
NW = NC * NS
CHUNK = 64
CPT = 158
E_PAD = CHUNK * CPT * NW
NCHT = E_PAD // CHUNK
CPT0 = 202
CPT1 = 2 * CPT - CPT0
N_PAD = 10112
RPT = N_PAD // NS


def _x1_body(x_ref, w_ref, o_ref):
    o_ref[...] = jnp.dot(x_ref[...], w_ref[...],
                         preferred_element_type=jnp.float32)


def _e2_body(e_ref, w_ref, b_ref, o_ref):
    r = jnp.dot(e_ref[...].astype(jnp.bfloat16), w_ref[...],
                preferred_element_type=jnp.float32) + b_ref[...]
    o_ref[...] = pltpu.bitcast(r.astype(jnp.bfloat16), jnp.int32)


def _apply_body(x_ref, h0_ref, h1_ref, wa1_ref, wa2_ref, b_ref, o_ref):
    acc = jnp.dot(x_ref[...], wa1_ref[...], preferred_element_type=jnp.float32)
    h = h0_ref[0] + h1_ref[0]
    acc = acc + jnp.dot(h, wa2_ref[...], preferred_element_type=jnp.float32)
    o_ref[...] = jnp.maximum(acc + b_ref[...], 0.0)


def _tc_x1(x, w1):
    nb = 10
    blk = N // nb
    return pl.pallas_call(
        _x1_body,
        grid=(nb,),
        in_specs=[
            pl.BlockSpec((blk, DIN), lambda i: (i, 0)),
            pl.BlockSpec((DIN, DOUT), lambda i: (0, 0)),
        ],
        out_specs=pl.BlockSpec((blk, DOUT), lambda i: (i, 0)),
        out_shape=jax.ShapeDtypeStruct((N, DOUT), jnp.float32),
    )(x, w1)


def _tc_e2(e, w2, b):
    blk = 2000
    nb = E // blk
    return pl.pallas_call(
        _e2_body,
        grid=(nb,),
        in_specs=[
            pl.BlockSpec((blk, DE), lambda i: (i, 0)),
            pl.BlockSpec((DE, DOUT), lambda i: (0, 0)),
            pl.BlockSpec((1, DOUT), lambda i: (0, 0)),
        ],
        out_specs=pl.BlockSpec((blk // 2, DOUT), lambda i: (i, 0)),
        out_shape=jax.ShapeDtypeStruct((E_PAD // 2, DOUT), jnp.int32),
    )(e, w2, b)


def _tc_apply(x, parts, wa1, wa2, b):
    nb = 10
    blk = N // nb
    return pl.pallas_call(
        _apply_body,
        grid=(nb,),
        in_specs=[
            pl.BlockSpec((blk, DIN), lambda i: (i, 0)),
            pl.BlockSpec((1, blk, DOUT), lambda i: (0, i, 0)),
            pl.BlockSpec((1, blk, DOUT), lambda i: (1, i, 0)),
            pl.BlockSpec((DIN, DOUT), lambda i: (0, 0)),
            pl.BlockSpec((DOUT, DOUT), lambda i: (0, 0)),
            pl.BlockSpec((1, DOUT), lambda i: (0, 0)),
        ],
        out_specs=pl.BlockSpec((blk, DOUT), lambda i: (i, 0)),
        out_shape=jax.ShapeDtypeStruct((N, DOUT), jnp.float32),
    )(x, parts, parts, wa1, wa2, b)


def _sc_body(x1_hbm, e2_hbm, idx_hbm, zeros_hbm, parts_hbm,
             idx0, idx1, rows0, rows1, e20, e21, sb0, sb1, dstx0, dstx1,
             acc_sh,
             lsem0, lsem1, gsem0, gsem1, esem0, esem1, ssem0, ssem1):
    c = lax.axis_index("c")
    s = lax.axis_index("s")
    idxb = (idx0, idx1)
    rows = (rows0, rows1)
    e2b = (e20, e21)
    sb = (sb0, sb1)
    dstx = (dstx0, dstx1)
    lsem = (lsem0, lsem1)
    gsem = (gsem0, gsem1)
    esem = (esem0, esem1)
    ssem = (ssem0, ssem1)

    cpt = CPT0 - (CPT0 - CPT1) * c
    cbase = c * (NS * CPT0) + s * cpt

    pltpu.sync_copy(zeros_hbm.at[pl.ds(s * RPT, RPT)],
                    acc_sh.at[pl.ds(s * RPT, RPT)])
    plsc.subcore_barrier()

    def issue_idx(p, i):
        pltpu.make_async_copy(idx_hbm.at[cbase + i], idxb[p], lsem[p]).start()

    def wait_idx(p):
        pltpu.make_async_copy(idx_hbm.at[0], idxb[p], lsem[p]).wait()

    def issue_gather(p):
        pltpu.make_async_copy(x1_hbm.at[idxb[p].at[0]], rows[p],
                              gsem[p]).start()

    def wait_gather(p):
        pltpu.make_async_copy(x1_hbm.at[idxb[p].at[0]], rows[p],
                              gsem[p]).wait()

    def issue_e2(p, i):
        off = (cbase + i) * (CHUNK // 2)
        pltpu.make_async_copy(e2_hbm.at[pl.ds(off, CHUNK // 2)], e2b[p],
                              esem[p]).start()

    def wait_e2(p):
        pltpu.make_async_copy(e2_hbm.at[pl.ds(0, CHUNK // 2)], e2b[p],
                              esem[p]).wait()

    def issue_scatter(p):
        pltpu.async_copy(sb[p], acc_sh.at[dstx[p]], ssem[p], add=True)

    def wait_scatter(p):
        pltpu.make_async_copy(sb[p], acc_sh.at[dstx[p]], ssem[p]).wait()

    def step(p, i):
        p1 = 1 - p
        wait_gather(p)

        @pl.when(i + 1 < cpt)
        def _():
            wait_idx(p1)
            issue_gather(p1)
            issue_e2(p1, i + 1)

        wait_e2(p)

        @pl.when(i >= 2)
        def _():
            wait_scatter(p)

        def cbody(r2):
            ra = 2 * r2
            rb = 2 * r2 + 1
            for j in range(8):
                sl = pl.ds(j * 16, 16)
                w = e2b[p][r2, sl]
                lo = lax.bitcast_convert_type(w << 16, jnp.float32)
                hi = lax.bitcast_convert_type(w & jnp.int32(-65536),
                                              jnp.float32)
                sb[p][ra, sl] = jnp.maximum(rows[p][ra, sl] + lo, 0.0)
                sb[p][rb, sl] = jnp.maximum(rows[p][rb, sl] + hi, 0.0)
        plsc.parallel_loop(0, CHUNK // 2, 1, unroll=2)(cbody)

        for j in range(4):
            sl = pl.ds(j * 16, 16)
            dstx[p][sl] = idxb[p][1, sl]
        issue_scatter(p)

        @pl.when(i + 2 < cpt)
        def _():
            issue_idx(p, i + 2)

    issue_idx(0, 0)
    issue_idx(1, 1)
    wait_idx(0)
    issue_gather(0)
    issue_e2(0, 0)

    def pair_body(k, _):
        step(0, 2 * k)
        step(1, 2 * k + 1)
        return 0

    lax.fori_loop(0, cpt // 2, pair_body, 0)
    wait_scatter(0)
    wait_scatter(1)
    plsc.subcore_barrier()

    pltpu.sync_copy(acc_sh.at[pl.ds(s * RPT, RPT)],
                    parts_hbm.at[c, pl.ds(s * RPT, RPT)])


def _sc_scatter(x1, e2p, idx_pairs, zeros):
    mesh = plsc.VectorSubcoreMesh(core_axis_name="c", subcore_axis_name="s")
    f = pl.kernel(
        _sc_body,
        out_type=jax.ShapeDtypeStruct((NC, N_PAD, DOUT), jnp.float32),
        mesh=mesh,
        scratch_types=[
            pltpu.VMEM((2, CHUNK), jnp.int32),
            pltpu.VMEM((2, CHUNK), jnp.int32),
            pltpu.VMEM((CHUNK, DOUT), jnp.float32),
            pltpu.VMEM((CHUNK, DOUT), jnp.float32),
            pltpu.VMEM((CHUNK // 2, DOUT), jnp.int32),
            pltpu.VMEM((CHUNK // 2, DOUT), jnp.int32),
            pltpu.VMEM((CHUNK, DOUT), jnp.float32),
            pltpu.VMEM((CHUNK, DOUT), jnp.float32),
            pltpu.VMEM((CHUNK,), jnp.int32),
            pltpu.VMEM((CHUNK,), jnp.int32),
            pltpu.VMEM_SHARED((N_PAD, DOUT), jnp.float32),
            pltpu.SemaphoreType.DMA,
            pltpu.SemaphoreType.DMA,
            pltpu.SemaphoreType.DMA,
            pltpu.SemaphoreType.DMA,
            pltpu.SemaphoreType.DMA,
            pltpu.SemaphoreType.DMA,
            pltpu.SemaphoreType.DMA,
            pltpu.SemaphoreType.DMA,
        ],
    )
    return f(x1, e2p, idx_pairs, zeros)


def kernel(edge_index, nfeats, efeats, W_msg, b_msg, W_apply, b_apply):
    x = nfeats.reshape(N, DIN)
    e = efeats.reshape(E, DE)
    src = edge_index[0]
    dst = edge_index[1]

    W1 = W_msg[:DIN]
    W2 = W_msg[DIN:]
    Wa1 = W_apply[:DIN]
    Wa2 = W_apply[DIN:]

    x1 = _tc_x1(x, W1)

    srcp = jnp.pad(src, (0, E_PAD - E))
    dstp = jnp.pad(dst, (0, E_PAD - E), constant_values=N)
    idx_pairs = jnp.stack([srcp.reshape(NCHT, CHUNK),
                           dstp.reshape(NCHT, CHUNK)], axis=1)
    zeros = jnp.zeros((N_PAD, DOUT), jnp.float32)

    e2p = _tc_e2(e, W2.astype(jnp.bfloat16),
                 b_msg.reshape(1, DOUT))
    parts = _sc_scatter(x1, e2p, idx_pairs, zeros)

    out = _tc_apply(x, parts, Wa1, Wa2, b_apply.reshape(1, DOUT))
    return out.reshape(N, 1, DOUT)


# speedup vs baseline: 4.5851x; 1.1749x over previous
import functools

import jax
import jax.numpy as jnp
from jax import lax
from jax.experimental import pallas as pl
from jax.experimental.pallas import tpu as pltpu
from jax.experimental.pallas import tpu_sc as plsc

N = 10000
E = 320000
DIN = 128
DE = 16
DOUT = 128

NC = 2
NS = 16
NW = NC * NS
CHUNK = 64
CPT = 158
E_PAD = CHUNK * CPT * NW
NCHT = E_PAD // CHUNK
CPT0 = 212
CPT1 = 2 * CPT - CPT0
N_PAD = 10112
RPT = N_PAD // NS


def _x1_body(x_ref, w_ref, o_ref):
    o_ref[...] = jnp.dot(x_ref[...], w_ref[...],
                         preferred_element_type=jnp.float32)


def _e2_body(e_ref, w_ref, b_ref, o_ref):
    r = jnp.dot(e_ref[...].astype(jnp.bfloat16), w_ref[...],
                preferred_element_type=jnp.float32) + b_ref[...]
    o_ref[...] = pltpu.bitcast(r.astype(jnp.bfloat16), jnp.int32)


def _apply_body(x_ref, h0_ref, h1_ref, wa1_ref, wa2_ref, b_ref, o_ref):
    acc = jnp.dot(x_ref[...], wa1_ref[...], preferred_element_type=jnp.float32)
    h = h0_ref[0] + h1_ref[0]
    acc = acc + jnp.dot(h, wa2_ref[...], preferred_element_type=jnp.float32)
    o_ref[...] = jnp.maximum(acc + b_ref[...], 0.0)


def _tc_x1(x, w1):
    nb = 10
    blk = N // nb
    return pl.pallas_call(
        _x1_body,
        grid=(nb,),
        in_specs=[
            pl.BlockSpec((blk, DIN), lambda i: (i, 0)),
            pl.BlockSpec((DIN, DOUT), lambda i: (0, 0)),
        ],
        out_specs=pl.BlockSpec((blk, DOUT), lambda i: (i, 0)),
        out_shape=jax.ShapeDtypeStruct((N, DOUT), jnp.float32),
    )(x, w1)


def _tc_e2(e, w2, b):
    blk = 8000
    nb = E // blk
    return pl.pallas_call(
        _e2_body,
        grid=(nb,),
        in_specs=[
            pl.BlockSpec((blk, DE), lambda i: (i, 0)),
            pl.BlockSpec((DE, DOUT), lambda i: (0, 0)),
            pl.BlockSpec((1, DOUT), lambda i: (0, 0)),
        ],
        out_specs=pl.BlockSpec((blk // 2, DOUT), lambda i: (i, 0)),
        out_shape=jax.ShapeDtypeStruct((E_PAD // 2, DOUT), jnp.int32),
    )(e, w2, b)


def _tc_apply(x, parts, wa1, wa2, b):
    nb = 10
    blk = N // nb
    return pl.pallas_call(
        _apply_body,
        grid=(nb,),
        in_specs=[
            pl.BlockSpec((blk, DIN), lambda i: (i, 0)),
            pl.BlockSpec((1, blk, DOUT), lambda i: (0, i, 0)),
            pl.BlockSpec((1, blk, DOUT), lambda i: (1, i, 0)),
            pl.BlockSpec((DIN, DOUT), lambda i: (0, 0)),
            pl.BlockSpec((DOUT, DOUT), lambda i: (0, 0)),
            pl.BlockSpec((1, DOUT), lambda i: (0, 0)),
        ],
        out_specs=pl.BlockSpec((blk, DOUT), lambda i: (i, 0)),
        out_shape=jax.ShapeDtypeStruct((N, DOUT), jnp.float32),
    )(x, parts, parts, wa1, wa2, b)


def _sc_body(x1_hbm, e2_hbm, idx_hbm, zeros_hbm, parts_hbm,
             idx0, idx1, rows0, rows1, e20, e21, sb0, sb1, dstx0, dstx1,
             acc_sh,
             lsem0, lsem1, gsem0, gsem1, esem0, esem1, ssem0, ssem1):
    c = lax.axis_index("c")
    s = lax.axis_index("s")
    idxb = (idx0, idx1)
    rows = (rows0, rows1)
    e2b = (e20, e21)
    sb = (sb0, sb1)
    dstx = (dstx0, dstx1)
    lsem = (lsem0, lsem1)
    gsem = (gsem0, gsem1)
    esem = (esem0, esem1)
    ssem = (ssem0, ssem1)

    cpt = CPT0 - (CPT0 - CPT1) * c
    cbase = c * (NS * CPT0) + s * cpt

    pltpu.sync_copy(zeros_hbm.at[pl.ds(s * RPT, RPT)],
                    acc_sh.at[pl.ds(s * RPT, RPT)])
    plsc.subcore_barrier()

    def issue_idx(p, i):
        pltpu.make_async_copy(idx_hbm.at[cbase + i], idxb[p], lsem[p]).start()

    def wait_idx(p):
        pltpu.make_async_copy(idx_hbm.at[0], idxb[p], lsem[p]).wait()

    def issue_gather(p):
        pltpu.make_async_copy(x1_hbm.at[idxb[p].at[0]], rows[p],
                              gsem[p]).start()

    def wait_gather(p):
        pltpu.make_async_copy(x1_hbm.at[idxb[p].at[0]], rows[p],
                              gsem[p]).wait()

    def issue_e2(p, i):
        off = (cbase + i) * (CHUNK // 2)
        pltpu.make_async_copy(e2_hbm.at[pl.ds(off, CHUNK // 2)], e2b[p],
                              esem[p]).start()

    def wait_e2(p):
        pltpu.make_async_copy(e2_hbm.at[pl.ds(0, CHUNK // 2)], e2b[p],
                              esem[p]).wait()

    def issue_scatter(p):
        pltpu.async_copy(sb[p], acc_sh.at[dstx[p]], ssem[p], add=True)

    def wait_scatter(p):
        pltpu.make_async_copy(sb[p], acc_sh.at[dstx[p]], ssem[p]).wait()

    def step(p, i):
        p1 = 1 - p
        wait_gather(p)

        @pl.when(i + 1 < cpt)
        def _():
            wait_idx(p1)
            issue_gather(p1)
            issue_e2(p1, i + 1)

        wait_e2(p)

        @pl.when(i >= 2)
        def _():
            wait_scatter(p)

        def cbody(r2):
            ra = 2 * r2
            rb = 2 * r2 + 1
            for j in range(8):
                sl = pl.ds(j * 16, 16)
                w = e2b[p][r2, sl]
                lo = lax.bitcast_convert_type(w << 16, jnp.float32)
                hi = lax.bitcast_convert_type(w & jnp.int32(-65536),
                                              jnp.float32)
                sb[p][ra, sl] = jnp.maximum(rows[p][ra, sl] + lo, 0.0)
                sb[p][rb, sl] = jnp.maximum(rows[p][rb, sl] + hi, 0.0)
        plsc.parallel_loop(0, CHUNK // 2, 1, unroll=2)(cbody)

        for j in range(4):
            sl = pl.ds(j * 16, 16)
            dstx[p][sl] = idxb[p][1, sl]
        issue_scatter(p)

        @pl.when(i + 2 < cpt)
        def _():
            issue_idx(p, i + 2)

    issue_idx(0, 0)
    issue_idx(1, 1)
    wait_idx(0)
    issue_gather(0)
    issue_e2(0, 0)

    def pair_body(k, _):
        step(0, 2 * k)
        step(1, 2 * k + 1)
        return 0

    lax.fori_loop(0, cpt // 2, pair_body, 0)
    wait_scatter(0)
    wait_scatter(1)
    plsc.subcore_barrier()

    pltpu.sync_copy(acc_sh.at[pl.ds(s * RPT, RPT)],
                    parts_hbm.at[c, pl.ds(s * RPT, RPT)])


def _sc_scatter(x1, e2p, idx_pairs, zeros):
    mesh = plsc.VectorSubcoreMesh(core_axis_name="c", subcore_axis_name="s")
    f = pl.kernel(
        _sc_body,
        out_type=jax.ShapeDtypeStruct((NC, N_PAD, DOUT), jnp.float32),
        mesh=mesh,
        scratch_types=[
            pltpu.VMEM((2, CHUNK), jnp.int32),
            pltpu.VMEM((2, CHUNK), jnp.int32),
            pltpu.VMEM((CHUNK, DOUT), jnp.float32),
            pltpu.VMEM((CHUNK, DOUT), jnp.float32),
            pltpu.VMEM((CHUNK // 2, DOUT), jnp.int32),
            pltpu.VMEM((CHUNK // 2, DOUT), jnp.int32),
            pltpu.VMEM((CHUNK, DOUT), jnp.float32),
            pltpu.VMEM((CHUNK, DOUT), jnp.float32),
            pltpu.VMEM((CHUNK,), jnp.int32),
            pltpu.VMEM((CHUNK,), jnp.int32),
            pltpu.VMEM_SHARED((N_PAD, DOUT), jnp.float32),
            pltpu.SemaphoreType.DMA,
            pltpu.SemaphoreType.DMA,
            pltpu.SemaphoreType.DMA,
            pltpu.SemaphoreType.DMA,
            pltpu.SemaphoreType.DMA,
            pltpu.SemaphoreType.DMA,
            pltpu.SemaphoreType.DMA,
            pltpu.SemaphoreType.DMA,
        ],
    )
    return f(x1, e2p, idx_pairs, zeros)


def kernel(edge_index, nfeats, efeats, W_msg, b_msg, W_apply, b_apply):
    x = nfeats.reshape(N, DIN)
    e = efeats.reshape(E, DE)
    src = edge_index[0]
    dst = edge_index[1]

    W1 = W_msg[:DIN]
    W2 = W_msg[DIN:]
    Wa1 = W_apply[:DIN]
    Wa2 = W_apply[DIN:]

    x1 = _tc_x1(x, W1)

    srcp = jnp.pad(src, (0, E_PAD - E))
    dstp = jnp.pad(dst, (0, E_PAD - E), constant_values=N)
    idx_pairs = jnp.stack([srcp.reshape(NCHT, CHUNK),
                           dstp.reshape(NCHT, CHUNK)], axis=1)
    zeros = jnp.zeros((N_PAD, DOUT), jnp.float32)

    e2p = _tc_e2(e, W2.astype(jnp.bfloat16),
                 b_msg.reshape(1, DOUT))
    parts = _sc_scatter(x1, e2p, idx_pairs, zeros)

    out = _tc_apply(x, parts, Wa1, Wa2, b_apply.reshape(1, DOUT))
    return out.reshape(N, 1, DOUT)


# speedup vs baseline: 4.7429x; 1.0344x over previous
import functools

import jax
import jax.numpy as jnp
from jax import lax
from jax.experimental import pallas as pl
from jax.experimental.pallas import tpu as pltpu
from jax.experimental.pallas import tpu_sc as plsc

N = 10000
E = 320000
DIN = 128
DE = 16
DOUT = 128

NC = 2
NS = 16
NW = NC * NS
CHUNK = 64
CPT = 158
E_PAD = CHUNK * CPT * NW
NCHT = E_PAD // CHUNK
CPT0 = 212
CPT1 = 2 * CPT - CPT0
N_PAD = 10112
RPT = N_PAD // NS


def _x1_body(x_ref, w_ref, o_ref):
    o_ref[...] = jnp.dot(x_ref[...], w_ref[...],
                         preferred_element_type=jnp.float32)


def _e2_body(e_ref, w_ref, b_ref, o_ref):
    r = jnp.dot(e_ref[...].astype(jnp.bfloat16), w_ref[...],
                preferred_element_type=jnp.float32) + b_ref[...]
    o_ref[...] = pltpu.bitcast(r.astype(jnp.bfloat16), jnp.int32)


def _apply_body(x_ref, h0_ref, h1_ref, wa1_ref, wa2_ref, b_ref, o_ref):
    acc = jnp.dot(x_ref[...], wa1_ref[...], preferred_element_type=jnp.float32)
    h = h0_ref[0] + h1_ref[0]
    acc = acc + jnp.dot(h, wa2_ref[...], preferred_element_type=jnp.float32)
    o_ref[...] = jnp.maximum(acc + b_ref[...], 0.0)


def _tc_x1(x, w1):
    nb = 10
    blk = N // nb
    return pl.pallas_call(
        _x1_body,
        grid=(nb,),
        in_specs=[
            pl.BlockSpec((blk, DIN), lambda i: (i, 0)),
            pl.BlockSpec((DIN, DOUT), lambda i: (0, 0)),
        ],
        out_specs=pl.BlockSpec((blk, DOUT), lambda i: (i, 0)),
        out_shape=jax.ShapeDtypeStruct((N, DOUT), jnp.float32),
    )(x, w1)


def _tc_e2(e, w2, b):
    blk = 8000
    nb = E // blk
    return pl.pallas_call(
        _e2_body,
        grid=(nb,),
        in_specs=[
            pl.BlockSpec((blk, DE), lambda i: (i, 0)),
            pl.BlockSpec((DE, DOUT), lambda i: (0, 0)),
            pl.BlockSpec((1, DOUT), lambda i: (0, 0)),
        ],
        out_specs=pl.BlockSpec((blk // 2, DOUT), lambda i: (i, 0)),
        out_shape=jax.ShapeDtypeStruct((E_PAD // 2, DOUT), jnp.int32),
    )(e, w2, b)


def _tc_apply(x, parts, wa1, wa2, b):
    nb = 10
    blk = N // nb
    return pl.pallas_call(
        _apply_body,
        grid=(nb,),
        in_specs=[
            pl.BlockSpec((blk, DIN), lambda i: (i, 0)),
            pl.BlockSpec((1, blk, DOUT), lambda i: (0, i, 0)),
            pl.BlockSpec((1, blk, DOUT), lambda i: (1, i, 0)),
            pl.BlockSpec((DIN, DOUT), lambda i: (0, 0)),
            pl.BlockSpec((DOUT, DOUT), lambda i: (0, 0)),
            pl.BlockSpec((1, DOUT), lambda i: (0, 0)),
        ],
        out_specs=pl.BlockSpec((blk, DOUT), lambda i: (i, 0)),
        out_shape=jax.ShapeDtypeStruct((N, DOUT), jnp.float32),
    )(x, parts, parts, wa1, wa2, b)


def _sc_body(x1_hbm, e2_hbm, idx_hbm, zeros_hbm, parts_hbm,
             idx0, idx1, rows0, rows1, e20, e21, sb0, sb1, dstx0, dstx1,
             acc_sh,
             lsem0, lsem1, gsem0, gsem1, esem0, esem1, ssem0, ssem1):
    c = lax.axis_index("c")
    s = lax.axis_index("s")
    idxb = (idx0, idx1)
    rows = (rows0, rows1)
    e2b = (e20, e21)
    sb = (sb0, sb1)
    dstx = (dstx0, dstx1)
    lsem = (lsem0, lsem1)
    gsem = (gsem0, gsem1)
    esem = (esem0, esem1)
    ssem = (ssem0, ssem1)

    cpt = CPT0 - (CPT0 - CPT1) * c
    cbase = c * (NS * CPT0) + s * cpt

    pltpu.sync_copy(zeros_hbm.at[pl.ds(s * RPT, RPT)],
                    acc_sh.at[pl.ds(s * RPT, RPT)])
    plsc.subcore_barrier()

    def issue_idx(p, i):
        pltpu.make_async_copy(idx_hbm.at[cbase + i], idxb[p], lsem[p]).start()

    def wait_idx(p):
        pltpu.make_async_copy(idx_hbm.at[0], idxb[p], lsem[p]).wait()

    def issue_gather(p):
        pltpu.make_async_copy(x1_hbm.at[idxb[p].at[pl.ds(0, CHUNK)]], rows[p],
                              gsem[p]).start()

    def wait_gather(p):
        pltpu.make_async_copy(x1_hbm.at[idxb[p].at[pl.ds(0, CHUNK)]], rows[p],
                              gsem[p]).wait()

    def issue_e2(p, i):
        off = (cbase + i) * (CHUNK // 2)
        pltpu.make_async_copy(e2_hbm.at[pl.ds(off, CHUNK // 2)], e2b[p],
                              esem[p]).start()

    def wait_e2(p):
        pltpu.make_async_copy(e2_hbm.at[pl.ds(0, CHUNK // 2)], e2b[p],
                              esem[p]).wait()

    def issue_scatter(p):
        pltpu.async_copy(sb[p], acc_sh.at[dstx[p]], ssem[p], add=True)

    def wait_scatter(p):
        pltpu.make_async_copy(sb[p], acc_sh.at[dstx[p]], ssem[p]).wait()

    def step(p, i):
        p1 = 1 - p
        wait_gather(p)

        @pl.when(i + 1 < cpt)
        def _():
            wait_idx(p1)
            issue_gather(p1)
            issue_e2(p1, i + 1)

        wait_e2(p)

        @pl.when(i >= 2)
        def _():
            wait_scatter(p)

        def cbody(r2):
            ra = 2 * r2
            rb = 2 * r2 + 1
            for j in range(8):
                sl = pl.ds(j * 16, 16)
                w = e2b[p][r2, sl]
                lo = lax.bitcast_convert_type(w << 16, jnp.float32)
                hi = lax.bitcast_convert_type(w & jnp.int32(-65536),
                                              jnp.float32)
                sb[p][ra, sl] = jnp.maximum(rows[p][ra, sl] + lo, 0.0)
                sb[p][rb, sl] = jnp.maximum(rows[p][rb, sl] + hi, 0.0)
        plsc.parallel_loop(0, CHUNK // 2, 1, unroll=2)(cbody)

        for j in range(4):
            dstx[p][pl.ds(j * 16, 16)] = idxb[p][pl.ds(CHUNK + j * 16, 16)]
        issue_scatter(p)

        @pl.when(i + 2 < cpt)
        def _():
            issue_idx(p, i + 2)

    issue_idx(0, 0)
    issue_idx(1, 1)
    wait_idx(0)
    issue_gather(0)
    issue_e2(0, 0)

    def pair_body(k, _):
        step(0, 2 * k)
        step(1, 2 * k + 1)
        return 0

    lax.fori_loop(0, cpt // 2, pair_body, 0)
    wait_scatter(0)
    wait_scatter(1)
    plsc.subcore_barrier()

    pltpu.sync_copy(acc_sh.at[pl.ds(s * RPT, RPT)],
                    parts_hbm.at[c, pl.ds(s * RPT, RPT)])


def _sc_scatter(x1, e2p, idx_pairs, zeros):
    mesh = plsc.VectorSubcoreMesh(core_axis_name="c", subcore_axis_name="s")
    f = pl.kernel(
        _sc_body,
        out_type=jax.ShapeDtypeStruct((NC, N_PAD, DOUT), jnp.float32),
        mesh=mesh,
        scratch_types=[
            pltpu.VMEM((2 * CHUNK,), jnp.int32),
            pltpu.VMEM((2 * CHUNK,), jnp.int32),
            pltpu.VMEM((CHUNK, DOUT), jnp.float32),
            pltpu.VMEM((CHUNK, DOUT), jnp.float32),
            pltpu.VMEM((CHUNK // 2, DOUT), jnp.int32),
            pltpu.VMEM((CHUNK // 2, DOUT), jnp.int32),
            pltpu.VMEM((CHUNK, DOUT), jnp.float32),
            pltpu.VMEM((CHUNK, DOUT), jnp.float32),
            pltpu.VMEM((CHUNK,), jnp.int32),
            pltpu.VMEM((CHUNK,), jnp.int32),
            pltpu.VMEM_SHARED((N_PAD, DOUT), jnp.float32),
            pltpu.SemaphoreType.DMA,
            pltpu.SemaphoreType.DMA,
            pltpu.SemaphoreType.DMA,
            pltpu.SemaphoreType.DMA,
            pltpu.SemaphoreType.DMA,
            pltpu.SemaphoreType.DMA,
            pltpu.SemaphoreType.DMA,
            pltpu.SemaphoreType.DMA,
        ],
    )
    return f(x1, e2p, idx_pairs, zeros)


def kernel(edge_index, nfeats, efeats, W_msg, b_msg, W_apply, b_apply):
    x = nfeats.reshape(N, DIN)
    e = efeats.reshape(E, DE)
    src = edge_index[0]
    dst = edge_index[1]

    W1 = W_msg[:DIN]
    W2 = W_msg[DIN:]
    Wa1 = W_apply[:DIN]
    Wa2 = W_apply[DIN:]

    x1 = _tc_x1(x, W1)

    srcp = jnp.pad(src, (0, E_PAD - E))
    dstp = jnp.pad(dst, (0, E_PAD - E), constant_values=N)
    idx_pairs = jnp.concatenate([srcp.reshape(NCHT, CHUNK),
                                 dstp.reshape(NCHT, CHUNK)], axis=1)
    zeros = jnp.zeros((N_PAD, DOUT), jnp.float32)

    e2p = _tc_e2(e, W2.astype(jnp.bfloat16),
                 b_msg.reshape(1, DOUT))
    parts = _sc_scatter(x1, e2p, idx_pairs, zeros)

    out = _tc_apply(x, parts, Wa1, Wa2, b_apply.reshape(1, DOUT))
    return out.reshape(N, 1, DOUT)


# speedup vs baseline: 4.7981x; 1.0116x over previous
import functools

import jax
import jax.numpy as jnp
from jax import lax
from jax.experimental import pallas as pl
from jax.experimental.pallas import tpu as pltpu
from jax.experimental.pallas import tpu_sc as plsc

N = 10000
E = 320000
DIN = 128
DE = 16
DOUT = 128

NC = 2
NS = 16
NW = NC * NS
CHUNK = 64
CPT = 158
E_PAD = CHUNK * CPT * NW
NCHT = E_PAD // CHUNK
CPT0 = 212
CPT1 = 2 * CPT - CPT0
N_PAD = 10112
RPT = N_PAD // NS


def _x1_body(x_ref, w_ref, o_ref):
    o_ref[...] = jnp.dot(x_ref[...], w_ref[...],
                         preferred_element_type=jnp.float32)


def _e2_body(e_ref, w_ref, b_ref, o_ref):
    r = jnp.dot(e_ref[...].astype(jnp.bfloat16), w_ref[...],
                preferred_element_type=jnp.float32) + b_ref[...]
    o_ref[...] = pltpu.bitcast(r.astype(jnp.bfloat16), jnp.int32)


def _apply_body(x_ref, h0_ref, h1_ref, wa1_ref, wa2_ref, b_ref, o_ref):
    acc = jnp.dot(x_ref[...], wa1_ref[...], preferred_element_type=jnp.float32)
    h = h0_ref[0] + h1_ref[0]
    acc = acc + jnp.dot(h, wa2_ref[...], preferred_element_type=jnp.float32)
    o_ref[...] = jnp.maximum(acc + b_ref[...], 0.0)


def _tc_x1(x, w1):
    nb = 10
    blk = N // nb
    return pl.pallas_call(
        _x1_body,
        grid=(nb,),
        in_specs=[
            pl.BlockSpec((blk, DIN), lambda i: (i, 0)),
            pl.BlockSpec((DIN, DOUT), lambda i: (0, 0)),
        ],
        out_specs=pl.BlockSpec((blk, DOUT), lambda i: (i, 0)),
        out_shape=jax.ShapeDtypeStruct((N, DOUT), jnp.float32),
    )(x, w1)


def _tc_e2(e, w2, b):
    blk = 16000
    nb = E // blk
    return pl.pallas_call(
        _e2_body,
        grid=(nb,),
        in_specs=[
            pl.BlockSpec((blk, DE), lambda i: (i, 0)),
            pl.BlockSpec((DE, DOUT), lambda i: (0, 0)),
            pl.BlockSpec((1, DOUT), lambda i: (0, 0)),
        ],
        out_specs=pl.BlockSpec((blk // 2, DOUT), lambda i: (i, 0)),
        out_shape=jax.ShapeDtypeStruct((E_PAD // 2, DOUT), jnp.int32),
    )(e, w2, b)


def _tc_apply(x, parts, wa1, wa2, b):
    nb = 10
    blk = N // nb
    return pl.pallas_call(
        _apply_body,
        grid=(nb,),
        in_specs=[
            pl.BlockSpec((blk, DIN), lambda i: (i, 0)),
            pl.BlockSpec((1, blk, DOUT), lambda i: (0, i, 0)),
            pl.BlockSpec((1, blk, DOUT), lambda i: (1, i, 0)),
            pl.BlockSpec((DIN, DOUT), lambda i: (0, 0)),
            pl.BlockSpec((DOUT, DOUT), lambda i: (0, 0)),
            pl.BlockSpec((1, DOUT), lambda i: (0, 0)),
        ],
        out_specs=pl.BlockSpec((blk, DOUT), lambda i: (i, 0)),
        out_shape=jax.ShapeDtypeStruct((N, DOUT), jnp.float32),
    )(x, parts, parts, wa1, wa2, b)


def _sc_body(x1_hbm, e2_hbm, idx_hbm, zeros_hbm, parts_hbm,
             idx0, idx1, rows0, rows1, e20, e21, sb0, sb1, dstx0, dstx1,
             acc_sh,
             lsem0, lsem1, gsem0, gsem1, esem0, esem1, ssem0, ssem1):
    c = lax.axis_index("c")
    s = lax.axis_index("s")
    idxb = (idx0, idx1)
    rows = (rows0, rows1)
    e2b = (e20, e21)
    sb = (sb0, sb1)
    dstx = (dstx0, dstx1)
    lsem = (lsem0, lsem1)
    gsem = (gsem0, gsem1)
    esem = (esem0, esem1)
    ssem = (ssem0, ssem1)

    cpt = CPT0 - (CPT0 - CPT1) * c
    cbase = c * (NS * CPT0) + s * cpt

    pltpu.sync_copy(zeros_hbm.at[pl.ds(s * RPT, RPT)],
                    acc_sh.at[pl.ds(s * RPT, RPT)])
    plsc.subcore_barrier()

    def issue_idx(p, i):
        pltpu.make_async_copy(idx_hbm.at[cbase + i], idxb[p], lsem[p]).start()

    def wait_idx(p):
        pltpu.make_async_copy(idx_hbm.at[0], idxb[p], lsem[p]).wait()

    def issue_gather(p):
        pltpu.make_async_copy(x1_hbm.at[idxb[p].at[pl.ds(0, CHUNK)]], rows[p],
                              gsem[p]).start()

    def wait_gather(p):
        pltpu.make_async_copy(x1_hbm.at[idxb[p].at[pl.ds(0, CHUNK)]], rows[p],
                              gsem[p]).wait()

    def issue_e2(p, i):
        off = (cbase + i) * (CHUNK // 2)
        pltpu.make_async_copy(e2_hbm.at[pl.ds(off, CHUNK // 2)], e2b[p],
                              esem[p]).start()

    def wait_e2(p):
        pltpu.make_async_copy(e2_hbm.at[pl.ds(0, CHUNK // 2)], e2b[p],
                              esem[p]).wait()

    def issue_scatter(p):
        pltpu.async_copy(sb[p], acc_sh.at[dstx[p]], ssem[p], add=True)

    def wait_scatter(p):
        pltpu.make_async_copy(sb[p], acc_sh.at[dstx[p]], ssem[p]).wait()

    def step(p, i):
        p1 = 1 - p
        wait_gather(p)

        @pl.when(i + 1 < cpt)
        def _():
            wait_idx(p1)
            issue_gather(p1)
            issue_e2(p1, i + 1)

        wait_e2(p)

        @pl.when(i >= 2)
        def _():
            wait_scatter(p)

        def cbody(r2):
            ra = 2 * r2
            rb = 2 * r2 + 1
            for j in range(8):
                sl = pl.ds(j * 16, 16)
                w = e2b[p][r2, sl]
                lo = lax.bitcast_convert_type(w << 16, jnp.float32)
                hi = lax.bitcast_convert_type(w & jnp.int32(-65536),
                                              jnp.float32)
                sb[p][ra, sl] = jnp.maximum(rows[p][ra, sl] + lo, 0.0)
                sb[p][rb, sl] = jnp.maximum(rows[p][rb, sl] + hi, 0.0)
        plsc.parallel_loop(0, CHUNK // 2, 1, unroll=2)(cbody)

        for j in range(4):
            dstx[p][pl.ds(j * 16, 16)] = idxb[p][pl.ds(CHUNK + j * 16, 16)]
        issue_scatter(p)

        @pl.when(i + 2 < cpt)
        def _():
            issue_idx(p, i + 2)

    issue_idx(0, 0)
    issue_idx(1, 1)
    wait_idx(0)
    issue_gather(0)
    issue_e2(0, 0)

    def pair_body(k, _):
        step(0, 2 * k)
        step(1, 2 * k + 1)
        return 0

    lax.fori_loop(0, cpt // 2, pair_body, 0)
    wait_scatter(0)
    wait_scatter(1)
    plsc.subcore_barrier()

    pltpu.sync_copy(acc_sh.at[pl.ds(s * RPT, RPT)],
                    parts_hbm.at[c, pl.ds(s * RPT, RPT)])


def _sc_scatter(x1, e2p, idx_pairs, zeros):
    mesh = plsc.VectorSubcoreMesh(core_axis_name="c", subcore_axis_name="s")
    f = pl.kernel(
        _sc_body,
        out_type=jax.ShapeDtypeStruct((NC, N_PAD, DOUT), jnp.float32),
        mesh=mesh,
        scratch_types=[
            pltpu.VMEM((2 * CHUNK,), jnp.int32),
            pltpu.VMEM((2 * CHUNK,), jnp.int32),
            pltpu.VMEM((CHUNK, DOUT), jnp.float32),
            pltpu.VMEM((CHUNK, DOUT), jnp.float32),
            pltpu.VMEM((CHUNK // 2, DOUT), jnp.int32),
            pltpu.VMEM((CHUNK // 2, DOUT), jnp.int32),
            pltpu.VMEM((CHUNK, DOUT), jnp.float32),
            pltpu.VMEM((CHUNK, DOUT), jnp.float32),
            pltpu.VMEM((CHUNK,), jnp.int32),
            pltpu.VMEM((CHUNK,), jnp.int32),
            pltpu.VMEM_SHARED((N_PAD, DOUT), jnp.float32),
            pltpu.SemaphoreType.DMA,
            pltpu.SemaphoreType.DMA,
            pltpu.SemaphoreType.DMA,
            pltpu.SemaphoreType.DMA,
            pltpu.SemaphoreType.DMA,
            pltpu.SemaphoreType.DMA,
            pltpu.SemaphoreType.DMA,
            pltpu.SemaphoreType.DMA,
        ],
    )
    return f(x1, e2p, idx_pairs, zeros)


def kernel(edge_index, nfeats, efeats, W_msg, b_msg, W_apply, b_apply):
    x = nfeats.reshape(N, DIN)
    e = efeats.reshape(E, DE)
    src = edge_index[0]
    dst = edge_index[1]

    W1 = W_msg[:DIN]
    W2 = W_msg[DIN:]
    Wa1 = W_apply[:DIN]
    Wa2 = W_apply[DIN:]

    x1 = _tc_x1(x, W1)

    srcp = jnp.pad(src, (0, E_PAD - E))
    dstp = jnp.pad(dst, (0, E_PAD - E), constant_values=N)
    idx_pairs = jnp.concatenate([srcp.reshape(NCHT, CHUNK),
                                 dstp.reshape(NCHT, CHUNK)], axis=1)
    zeros = jnp.zeros((N_PAD, DOUT), jnp.float32)

    e2p = _tc_e2(e, W2.astype(jnp.bfloat16),
                 b_msg.reshape(1, DOUT))
    parts = _sc_scatter(x1, e2p, idx_pairs, zeros)

    out = _tc_apply(x, parts, Wa1, Wa2, b_apply.reshape(1, DOUT))
    return out.reshape(N, 1, DOUT)
